# baseline (speedup 1.0000x reference)
import functools

import jax
import jax.numpy as jnp
from jax import lax
from jax.experimental import pallas as pl
from jax.experimental.pallas import tpu as pltpu
from jax.experimental.pallas import tpu_sc as plsc

B = 8
N = 4096
CHK = 128


def _fps_body(x_ref, o_ref, *, n, npoint):
    x = x_ref[:, 0, :]
    y = x_ref[:, 1, :]
    z = x_ref[:, 2, :]
    iota_n = lax.broadcasted_iota(jnp.int32, (B, n), 1)
    iota_p = lax.broadcasted_iota(jnp.int32, (B, npoint), 1)
    dist0 = jnp.full((B, n), 1e10, jnp.float32)
    far0 = jnp.zeros((B, 1), jnp.int32)
    acc0 = jnp.zeros((B, npoint), jnp.float32)

    def body(i, st):
        dist, far, ax, ay, az = st
        sel = (iota_n == far).astype(jnp.float32)
        cx = jnp.sum(x * sel, axis=1, keepdims=True)
        cy = jnp.sum(y * sel, axis=1, keepdims=True)
        cz = jnp.sum(z * sel, axis=1, keepdims=True)
        oh = (iota_p == i).astype(jnp.float32)
        ax = ax + oh * cx
        ay = ay + oh * cy
        az = az + oh * cz
        dx = x - cx
        dy = y - cy
        dz = z - cz
        d = dx * dx + dy * dy + dz * dz
        dist = jnp.minimum(dist, d)
        m = jnp.max(dist, axis=1, keepdims=True)
        far = jnp.min(jnp.where(dist == m, iota_n, n), axis=1, keepdims=True)
        return dist, far, ax, ay, az

    _, _, ax, ay, az = lax.fori_loop(0, npoint, body,
                                     (dist0, far0, acc0, acc0, acc0))
    o_ref[:, 0, :] = ax
    o_ref[:, 1, :] = ay
    o_ref[:, 2, :] = az


def _fps(xyz_t, n, npoint):
    return pl.pallas_call(
        functools.partial(_fps_body, n=n, npoint=npoint),
        out_shape=jax.ShapeDtypeStruct((B, 3, npoint), jnp.float32),
    )(xyz_t)


def _radius_step(j, sq, ut, o_ref, car_ref, r2):
    @pl.when(j == 0)
    def _():
        car_ref[...] = jnp.zeros_like(car_ref)

    maskf = (sq <= r2).astype(jnp.float32)
    cum = jnp.dot(maskf, ut, preferred_element_type=jnp.float32)
    carry = car_ref[...]
    o_ref[0] = cum + carry
    last = cum[:, cum.shape[1] - 1:]
    car_ref[...] = carry + jnp.broadcast_to(last, carry.shape)


def _rank_body(x_ref, c_ref, o0, o1, o2, car0, car1, car2, *, r2s):
    j = pl.program_id(1)
    px = x_ref[0, 0:1, :]
    py = x_ref[0, 1:2, :]
    pz = x_ref[0, 2:3, :]
    cx = c_ref[0, :, 0:1]
    cy = c_ref[0, :, 1:2]
    cz = c_ref[0, :, 2:3]
    dx = cx - px
    dy = cy - py
    dz = cz - pz
    sq = dx * dx + dy * dy + dz * dz
    row = lax.broadcasted_iota(jnp.int32, (CHK, CHK), 0)
    col = lax.broadcasted_iota(jnp.int32, (CHK, CHK), 1)
    ut = (row <= col).astype(jnp.float32)
    _radius_step(j, sq, ut, o0, car0, r2s[0])
    _radius_step(j, sq, ut, o1, car1, r2s[1])
    _radius_step(j, sq, ut, o2, car2, r2s[2])


def _rank(xyz_t, cent_rows, radii, np_, n):
    nchunk = n // CHK
    r2s = tuple(float(r) * float(r) for r in radii)
    outs = pl.pallas_call(
        functools.partial(_rank_body, r2s=r2s),
        grid=(B, nchunk),
        in_specs=[
            pl.BlockSpec((1, 3, CHK), lambda b, j: (b, 0, j)),
            pl.BlockSpec((1, np_, 3), lambda b, j: (b, 0, 0)),
        ],
        out_specs=[pl.BlockSpec((1, np_, CHK), lambda b, j: (b, 0, j))] * 3,
        out_shape=[jax.ShapeDtypeStruct((B, np_, n), jnp.float32)] * 3,
        scratch_shapes=[pltpu.VMEM((np_, CHK), jnp.float32)] * 3,
    )(xyz_t, cent_rows)
    return outs


def _make_sc_gather(nrows, npts, ns, c, rows_per_b, tab_rows, logn):
    mesh = plsc.VectorSubcoreMesh(core_axis_name="c", subcore_axis_name="s")
    rows_per_w = nrows // 32

    @functools.partial(
        pl.kernel,
        mesh=mesh,
        out_type=jax.ShapeDtypeStruct((nrows * ns, c), jnp.float32),
        scratch_types=[
            pltpu.VMEM((npts,), jnp.float32),
            pltpu.VMEM((ns,), jnp.int32),
            pltpu.VMEM((ns, c), jnp.float32),
            pltpu.SemaphoreType.DMA,
        ],
        compiler_params=pltpu.CompilerParams(needs_layout_passes=False,
                                             use_tc_tiling_on_sc=False),
    )
    def k(rank_hbm, table_hbm, out_hbm, rank_v, idx_v, rows_v, sem):
        wid = lax.axis_index("s") * 2 + lax.axis_index("c")

        def row_step(t, carry):
            r = wid * rows_per_w + t
            pltpu.sync_copy(rank_hbm.at[r], rank_v)
            base = (r // rows_per_b) * tab_rows
            first = jnp.zeros((16,), jnp.int32)
            for gi in range(ns // 16):
                ks = lax.iota(jnp.int32, 16) + (gi * 16)
                target = (ks + 1).astype(jnp.float32)
                lo = jnp.zeros((16,), jnp.int32)
                hi = jnp.full((16,), npts, jnp.int32)
                for _ in range(logn):
                    mid = jnp.minimum((lo + hi) // 2, npts - 1)
                    vv = plsc.load_gather(rank_v, [mid])
                    ge = vv >= target
                    hi = jnp.where(ge, mid, hi)
                    lo = jnp.where(ge, lo, mid + 1)
                if gi == 0:
                    first = jnp.full((16,), jnp.min(lo), jnp.int32)
                idx = jnp.where(lo >= npts, first, lo) + base
                idx_v[pl.ds(gi * 16, 16)] = idx
            pltpu.async_copy(table_hbm.at[idx_v], rows_v, sem).wait()
            pltpu.sync_copy(rows_v, out_hbm.at[pl.ds(r * ns, ns)])
            return carry

        lax.fori_loop(0, rows_per_w, row_step, 0)

    return k


def _k1_body(x_ref, c_ref, w_ref, b_ref, y_ref, s_ref, ss_ref, *, mb, ns):
    @pl.when(pl.program_id(0) == 0)
    def _():
        s_ref[...] = jnp.zeros_like(s_ref)
        ss_ref[...] = jnp.zeros_like(ss_ref)

    ct = x_ref.shape[2]
    xc = x_ref[...] - c_ref[...][:, None, :]
    x2 = xc.reshape(mb * ns, ct)
    y2 = jnp.dot(x2, w_ref[...], preferred_element_type=jnp.float32) + b_ref[...]
    y_ref[...] = y2.reshape(mb, ns, y2.shape[1])
    s_ref[...] += jnp.broadcast_to(jnp.sum(y2, axis=0, keepdims=True), s_ref.shape)
    ss_ref[...] += jnp.broadcast_to(jnp.sum(y2 * y2, axis=0, keepdims=True), ss_ref.shape)


def _layer1(x, cent, w, b, mb, ns):
    m = x.shape[0]
    ct = x.shape[2]
    cout = w.shape[1]
    return pl.pallas_call(
        functools.partial(_k1_body, mb=mb, ns=ns),
        grid=(m // mb,),
        in_specs=[
            pl.BlockSpec((mb, ns, ct), lambda i: (i, 0, 0)),
            pl.BlockSpec((mb, ct), lambda i: (i, 0)),
            pl.BlockSpec(w.shape, lambda i: (0, 0)),
            pl.BlockSpec((1, cout), lambda i: (0, 0)),
        ],
        out_specs=[
            pl.BlockSpec((mb, ns, cout), lambda i: (i, 0, 0)),
            pl.BlockSpec((8, cout), lambda i: (0, 0)),
            pl.BlockSpec((8, cout), lambda i: (0, 0)),
        ],
        out_shape=[
            jax.ShapeDtypeStruct((m, ns, cout), jnp.float32),
            jax.ShapeDtypeStruct((8, cout), jnp.float32),
            jax.ShapeDtypeStruct((8, cout), jnp.float32),
        ],
    )(x, cent, w, b)


def _k1b_body(x_ref, c_ref, b_ref, y_ref, s_ref, ss_ref, *, mb, ns):
    @pl.when(pl.program_id(0) == 0)
    def _():
        s_ref[...] = jnp.zeros_like(s_ref)
        ss_ref[...] = jnp.zeros_like(ss_ref)

    y = x_ref[...] - c_ref[...][:, None, :] + b_ref[...][None]
    y_ref[...] = y
    y2 = y.reshape(mb * ns, y.shape[2])
    s_ref[...] += jnp.broadcast_to(jnp.sum(y2, axis=0, keepdims=True), s_ref.shape)
    ss_ref[...] += jnp.broadcast_to(jnp.sum(y2 * y2, axis=0, keepdims=True), ss_ref.shape)


def _layer1b(x, cent, b, mb, ns):
    m = x.shape[0]
    cout = x.shape[2]
    return pl.pallas_call(
        functools.partial(_k1b_body, mb=mb, ns=ns),
        grid=(m // mb,),
        in_specs=[
            pl.BlockSpec((mb, ns, cout), lambda i: (i, 0, 0)),
            pl.BlockSpec((mb, cout), lambda i: (i, 0)),
            pl.BlockSpec((1, cout), lambda i: (0, 0)),
        ],
        out_specs=[
            pl.BlockSpec((mb, ns, cout), lambda i: (i, 0, 0)),
            pl.BlockSpec((8, cout), lambda i: (0, 0)),
            pl.BlockSpec((8, cout), lambda i: (0, 0)),
        ],
        out_shape=[
            jax.ShapeDtypeStruct((m, ns, cout), jnp.float32),
            jax.ShapeDtypeStruct((8, cout), jnp.float32),
            jax.ShapeDtypeStruct((8, cout), jnp.float32),
        ],
    )(x, cent, b)


def _kmid_body(x_ref, si_ref, ssi_ref, g_ref, be_ref, w_ref, b_ref,
               y_ref, s_ref, ss_ref, *, mb, ns, cnt):
    @pl.when(pl.program_id(0) == 0)
    def _():
        s_ref[...] = jnp.zeros_like(s_ref)
        ss_ref[...] = jnp.zeros_like(ss_ref)

    cin = x_ref.shape[2]
    mean = si_ref[0:1, :] / cnt
    var = ssi_ref[0:1, :] / cnt - mean * mean
    rs = lax.rsqrt(var + 1e-5)
    x2 = x_ref[...].reshape(mb * ns, cin)
    xn = jnp.maximum((x2 - mean) * rs * g_ref[...] + be_ref[...], 0.0)
    y2 = jnp.dot(xn, w_ref[...], preferred_element_type=jnp.float32) + b_ref[...]
    y_ref[...] = y2.reshape(mb, ns, y2.shape[1])
    s_ref[...] += jnp.broadcast_to(jnp.sum(y2, axis=0, keepdims=True), s_ref.shape)
    ss_ref[...] += jnp.broadcast_to(jnp.sum(y2 * y2, axis=0, keepdims=True), ss_ref.shape)


def _layermid(x, si, ssi, g, be, w, b, mb, ns, cnt):
    m = x.shape[0]
    cin = x.shape[2]
    cout = w.shape[1]
    return pl.pallas_call(
        functools.partial(_kmid_body, mb=mb, ns=ns, cnt=float(cnt)),
        grid=(m // mb,),
        in_specs=[
            pl.BlockSpec((mb, ns, cin), lambda i: (i, 0, 0)),
            pl.BlockSpec((8, cin), lambda i: (0, 0)),
            pl.BlockSpec((8, cin), lambda i: (0, 0)),
            pl.BlockSpec((1, cin), lambda i: (0, 0)),
            pl.BlockSpec((1, cin), lambda i: (0, 0)),
            pl.BlockSpec(w.shape, lambda i: (0, 0)),
            pl.BlockSpec((1, cout), lambda i: (0, 0)),
        ],
        out_specs=[
            pl.BlockSpec((mb, ns, cout), lambda i: (i, 0, 0)),
            pl.BlockSpec((8, cout), lambda i: (0, 0)),
            pl.BlockSpec((8, cout), lambda i: (0, 0)),
        ],
        out_shape=[
            jax.ShapeDtypeStruct((m, ns, cout), jnp.float32),
            jax.ShapeDtypeStruct((8, cout), jnp.float32),
            jax.ShapeDtypeStruct((8, cout), jnp.float32),
        ],
    )(x, si, ssi, g.reshape(1, cin), be.reshape(1, cin), w, b)


def _kfin_body(x_ref, si_ref, ssi_ref, g_ref, be_ref, o_ref, *, mb, ns, cnt):
    cin = x_ref.shape[2]
    mean = si_ref[0:1, :] / cnt
    var = ssi_ref[0:1, :] / cnt - mean * mean
    rs = lax.rsqrt(var + 1e-5)
    x2 = x_ref[...].reshape(mb * ns, cin)
    xn = jnp.maximum((x2 - mean) * rs * g_ref[...] + be_ref[...], 0.0)
    o_ref[...] = jnp.max(xn.reshape(mb, ns, cin), axis=1)


def _layerfin(x, si, ssi, g, be, mb, ns, cnt):
    m = x.shape[0]
    cin = x.shape[2]
    return pl.pallas_call(
        functools.partial(_kfin_body, mb=mb, ns=ns, cnt=float(cnt)),
        grid=(m // mb,),
        in_specs=[
            pl.BlockSpec((mb, ns, cin), lambda i: (i, 0, 0)),
            pl.BlockSpec((8, cin), lambda i: (0, 0)),
            pl.BlockSpec((8, cin), lambda i: (0, 0)),
            pl.BlockSpec((1, cin), lambda i: (0, 0)),
            pl.BlockSpec((1, cin), lambda i: (0, 0)),
        ],
        out_specs=pl.BlockSpec((mb, cin), lambda i: (i, 0)),
        out_shape=jax.ShapeDtypeStruct((m, cin), jnp.float32),
    )(x, si, ssi, g.reshape(1, cin), be.reshape(1, cin))


def _mm_body(x_ref, w_ref, o_ref):
    o_ref[...] = jnp.dot(x_ref[...], w_ref[...],
                         preferred_element_type=jnp.float32)


def _mm(x, w):
    return pl.pallas_call(
        _mm_body,
        out_shape=jax.ShapeDtypeStruct((x.shape[0], w.shape[1]), jnp.float32),
    )(x, w)


def _fc_body(x_ref, w1, b1, w2, b2, w3, b3, o_ref):
    h = jnp.dot(x_ref[...], w1[...], preferred_element_type=jnp.float32) + b1[...]
    h = jnp.dot(h, w2[...], preferred_element_type=jnp.float32) + b2[...]
    o_ref[...] = jnp.dot(h, w3[...], preferred_element_type=jnp.float32) + b3[...]


def _fc(x, fc1, fc2, fc3):
    return pl.pallas_call(
        _fc_body,
        out_shape=jax.ShapeDtypeStruct((x.shape[0], fc3[0].shape[1]), jnp.float32),
    )(x, fc1[0], fc1[1].reshape(1, -1), fc2[0], fc2[1].reshape(1, -1),
      fc3[0], fc3[1].reshape(1, -1))


def _msg_scale(gathered, cent, layers, first_is_mm, w1p, mb, ns, m):
    (w1, b1, g1, be1) = layers[0]
    if first_is_mm:
        y1, s1, ss1 = _layer1(gathered, cent, w1p, b1.reshape(1, -1), mb, ns)
    else:
        y1, s1, ss1 = _layer1b(gathered, cent, b1.reshape(1, -1), mb, ns)
    (w2, b2, g2, be2) = layers[1]
    y2, s2, ss2 = _layermid(y1, s1, ss1, g1, be1, w2, b2.reshape(1, -1),
                            mb, ns, m * ns)
    (w3, b3, g3, be3) = layers[2]
    y3, s3, ss3 = _layermid(y2, s2, ss2, g2, be2, w3, b3.reshape(1, -1),
                            mb, ns, m * ns)
    return _layerfin(y3, s3, ss3, g3, be3, mb, ns, m * ns)


def kernel(pointcloud, params):
    pc = pointcloud.astype(jnp.float32)
    xyz = pc[..., :3]
    xyz_t = jnp.transpose(xyz, (0, 2, 1))

    cent1_t = _fps(xyz_t, N, 512)
    cent1_rows = jnp.transpose(cent1_t, (0, 2, 1))
    rank1 = _rank(xyz_t, cent1_rows, (0.1, 0.2, 0.4), np_=512, n=N)
    table1 = jnp.pad(pc.reshape(B * N, 6), ((0, 0), (0, 10)))
    cent1_pad = jnp.pad(cent1_rows.reshape(B * 512, 3), ((0, 0), (0, 13)))
    outs1 = []
    for i, ns in enumerate((16, 32, 128)):
        sck = _make_sc_gather(nrows=B * 512, npts=N, ns=ns, c=16,
                              rows_per_b=512, tab_rows=N, logn=13)
        g = sck(rank1[i].reshape(B * 512, N), table1)
        g = g.reshape(B * 512, ns, 16)
        w1p = jnp.pad(params["sa1"][i][0][0], ((0, 10), (0, 0)))
        mb = 4096 // ns if ns > 16 else 256
        outs1.append(_msg_scale(g, cent1_pad, params["sa1"][i], True, w1p,
                                mb=mb, ns=ns, m=B * 512))
    feats1 = jnp.concatenate([o.reshape(B, 512, o.shape[1]) for o in outs1], -1)

    x2 = jnp.concatenate([cent1_rows, feats1], -1).reshape(B * 512, 323)
    cent2_t = _fps(cent1_t, 512, 128)
    cent2_rows = jnp.transpose(cent2_t, (0, 2, 1))
    rank2 = _rank(cent1_t, cent2_rows, (0.2, 0.4, 0.8), np_=128, n=512)
    cent2_pad = jnp.pad(cent2_rows.reshape(B * 128, 3), ((0, 0), (0, 5)))
    outs2 = []
    for i, ns in enumerate((16, 32, 128)):
        w1 = params["sa2"][i][0][0]
        c1 = w1.shape[1]
        u = _mm(x2, w1)
        w1xyz = jnp.pad(w1[:3], ((0, 5), (0, 0)))
        cu = _mm(cent2_pad, w1xyz)
        sck = _make_sc_gather(nrows=B * 128, npts=512, ns=ns, c=c1,
                              rows_per_b=128, tab_rows=512, logn=10)
        g = sck(rank2[i].reshape(B * 128, 512), u)
        g = g.reshape(B * 128, ns, c1)
        mb = 4096 // ns if ns > 16 else 256
        outs2.append(_msg_scale(g, cu, params["sa2"][i], False, None,
                                mb=mb, ns=ns, m=B * 128))
    feats2 = jnp.concatenate([o.reshape(B, 128, o.shape[1]) for o in outs2], -1)

    x3 = jnp.concatenate([cent2_rows, feats2], -1)
    l3 = params["sa3"]
    zc = jnp.zeros((B, x3.shape[2]), jnp.float32)
    y1, s1, ss1 = _layer1(x3, zc, l3[0][0], l3[0][1].reshape(1, -1), mb=B, ns=128)
    y2, s2, ss2 = _layermid(y1, s1, ss1, l3[0][2], l3[0][3], l3[1][0],
                            l3[1][1].reshape(1, -1), mb=B, ns=128, cnt=B * 128)
    y3, s3, ss3 = _layermid(y2, s2, ss2, l3[1][2], l3[1][3], l3[2][0],
                            l3[2][1].reshape(1, -1), mb=B, ns=128, cnt=B * 128)
    f = _layerfin(y3, s3, ss3, l3[2][2], l3[2][3], mb=B, ns=128, cnt=B * 128)
    return _fc(f, params["fc1"], params["fc2"], params["fc3"])


# speedup vs baseline: 1.1098x; 1.1098x over previous
import functools

import jax
import jax.numpy as jnp
from jax import lax
from jax.experimental import pallas as pl
from jax.experimental.pallas import tpu as pltpu
from jax.experimental.pallas import tpu_sc as plsc

B = 8
N = 4096
CHK = 128


def _fps_body(x_ref, o_ref, *, n, npoint):
    x = x_ref[:, 0, :]
    y = x_ref[:, 1, :]
    z = x_ref[:, 2, :]
    iota_n = lax.broadcasted_iota(jnp.int32, (B, n), 1)
    iota_p = lax.broadcasted_iota(jnp.int32, (B, npoint), 1)
    dist0 = jnp.full((B, n), 1e10, jnp.float32)
    far0 = jnp.zeros((B, 1), jnp.int32)
    acc0 = jnp.zeros((B, npoint), jnp.float32)

    def body(i, st):
        dist, far, ax, ay, az = st
        sel = (iota_n == far).astype(jnp.float32)
        cx = jnp.sum(x * sel, axis=1, keepdims=True)
        cy = jnp.sum(y * sel, axis=1, keepdims=True)
        cz = jnp.sum(z * sel, axis=1, keepdims=True)
        oh = (iota_p == i).astype(jnp.float32)
        ax = ax + oh * cx
        ay = ay + oh * cy
        az = az + oh * cz
        dx = x - cx
        dy = y - cy
        dz = z - cz
        d = dx * dx + dy * dy + dz * dz
        dist = jnp.minimum(dist, d)
        m = jnp.max(dist, axis=1, keepdims=True)
        far = jnp.min(jnp.where(dist == m, iota_n, n), axis=1, keepdims=True)
        return dist, far, ax, ay, az

    _, _, ax, ay, az = lax.fori_loop(0, npoint, body,
                                     (dist0, far0, acc0, acc0, acc0))
    o_ref[:, 0, :] = ax
    o_ref[:, 1, :] = ay
    o_ref[:, 2, :] = az


def _fps(xyz_t, n, npoint):
    return pl.pallas_call(
        functools.partial(_fps_body, n=n, npoint=npoint),
        out_shape=jax.ShapeDtypeStruct((B, 3, npoint), jnp.float32),
    )(xyz_t)


def _radius_step(j, sq, ut, o_ref, car_ref, r2):
    @pl.when(j == 0)
    def _():
        car_ref[...] = jnp.zeros_like(car_ref)

    maskf = (sq <= r2).astype(jnp.float32)
    cum = jnp.dot(maskf, ut, preferred_element_type=jnp.float32)
    carry = car_ref[...]
    o_ref[0] = cum + carry
    last = cum[:, cum.shape[1] - 1:]
    car_ref[...] = carry + jnp.broadcast_to(last, carry.shape)


def _rank_body(x_ref, c_ref, o0, o1, o2, car0, car1, car2, *, r2s):
    j = pl.program_id(1)
    px = x_ref[0, 0:1, :]
    py = x_ref[0, 1:2, :]
    pz = x_ref[0, 2:3, :]
    cx = c_ref[0, :, 0:1]
    cy = c_ref[0, :, 1:2]
    cz = c_ref[0, :, 2:3]
    dx = cx - px
    dy = cy - py
    dz = cz - pz
    sq = dx * dx + dy * dy + dz * dz
    row = lax.broadcasted_iota(jnp.int32, (CHK, CHK), 0)
    col = lax.broadcasted_iota(jnp.int32, (CHK, CHK), 1)
    ut = (row <= col).astype(jnp.float32)
    _radius_step(j, sq, ut, o0, car0, r2s[0])
    _radius_step(j, sq, ut, o1, car1, r2s[1])
    _radius_step(j, sq, ut, o2, car2, r2s[2])


def _rank(xyz_t, cent_rows, radii, np_, n):
    nchunk = n // CHK
    r2s = tuple(float(r) * float(r) for r in radii)
    outs = pl.pallas_call(
        functools.partial(_rank_body, r2s=r2s),
        grid=(B, nchunk),
        in_specs=[
            pl.BlockSpec((1, 3, CHK), lambda b, j: (b, 0, j)),
            pl.BlockSpec((1, np_, 3), lambda b, j: (b, 0, 0)),
        ],
        out_specs=[pl.BlockSpec((1, np_, CHK), lambda b, j: (b, 0, j))] * 3,
        out_shape=[jax.ShapeDtypeStruct((B, np_, n), jnp.float32)] * 3,
        scratch_shapes=[pltpu.VMEM((np_, CHK), jnp.float32)] * 3,
    )(xyz_t, cent_rows)
    return outs


def _make_sc_gather(nrows, npts, ns, c, rows_per_b, tab_rows, logn):
    mesh = plsc.VectorSubcoreMesh(core_axis_name="c", subcore_axis_name="s")
    rows_per_w = nrows // 32

    @functools.partial(
        pl.kernel,
        mesh=mesh,
        out_type=jax.ShapeDtypeStruct((nrows * ns, c), jnp.float32),
        scratch_types=[
            pltpu.VMEM((2, npts), jnp.float32),
            pltpu.VMEM((ns,), jnp.int32),
            pltpu.VMEM((2, ns, c), jnp.float32),
            pltpu.SemaphoreType.DMA,
            pltpu.SemaphoreType.DMA,
            pltpu.SemaphoreType.DMA,
        ],
        compiler_params=pltpu.CompilerParams(needs_layout_passes=False,
                                             use_tc_tiling_on_sc=False),
    )
    def k(rank_hbm, table_hbm, out_hbm, rank_v, idx_v, rows_v, sem_r, sem_g,
          sem_o):
        wid = lax.axis_index("s") * 2 + lax.axis_index("c")
        r0 = wid * rows_per_w
        pltpu.async_copy(rank_hbm.at[r0], rank_v.at[0], sem_r)

        def row_step(t, carry):
            r = r0 + t
            buf = t % 2
            pltpu.make_async_copy(rank_hbm.at[r], rank_v.at[buf], sem_r).wait()

            @pl.when(t + 1 < rows_per_w)
            def _():
                pltpu.async_copy(rank_hbm.at[r + 1], rank_v.at[1 - buf], sem_r)

            base = (r // rows_per_b) * tab_rows
            bufv = jnp.full((16,), buf, jnp.int32)
            first = jnp.zeros((16,), jnp.int32)
            for gi in range(ns // 16):
                ks = lax.iota(jnp.int32, 16) + (gi * 16)
                target = (ks + 1).astype(jnp.float32)
                lo = jnp.zeros((16,), jnp.int32)
                hi = jnp.full((16,), npts, jnp.int32)
                for _ in range(logn):
                    mid = jnp.minimum((lo + hi) // 2, npts - 1)
                    vv = plsc.load_gather(rank_v, [bufv, mid])
                    ge = vv >= target
                    hi = jnp.where(ge, mid, hi)
                    lo = jnp.where(ge, lo, mid + 1)
                if gi == 0:
                    first = jnp.full((16,), jnp.min(lo), jnp.int32)
                idx = jnp.where(lo >= npts, first, lo) + base
                idx_v[pl.ds(gi * 16, 16)] = idx

            @pl.when(t >= 2)
            def _():
                pltpu.make_async_copy(
                    rows_v.at[buf], out_hbm.at[pl.ds(r * ns, ns)], sem_o
                ).wait()

            pltpu.async_copy(table_hbm.at[idx_v], rows_v.at[buf], sem_g).wait()
            pltpu.async_copy(rows_v.at[buf], out_hbm.at[pl.ds(r * ns, ns)],
                             sem_o)
            return carry

        lax.fori_loop(0, rows_per_w, row_step, 0)
        pltpu.make_async_copy(rows_v.at[0], out_hbm.at[pl.ds(0, ns)],
                              sem_o).wait()
        pltpu.make_async_copy(rows_v.at[1], out_hbm.at[pl.ds(0, ns)],
                              sem_o).wait()

    return k


def _k1_body(x_ref, c_ref, w_ref, b_ref, y_ref, s_ref, ss_ref, *, mb, ns):
    @pl.when(pl.program_id(0) == 0)
    def _():
        s_ref[...] = jnp.zeros_like(s_ref)
        ss_ref[...] = jnp.zeros_like(ss_ref)

    ct = x_ref.shape[2]
    xc = x_ref[...] - c_ref[...][:, None, :]
    x2 = xc.reshape(mb * ns, ct)
    y2 = jnp.dot(x2, w_ref[...], preferred_element_type=jnp.float32) + b_ref[...]
    y_ref[...] = y2.reshape(mb, ns, y2.shape[1])
    s_ref[...] += jnp.broadcast_to(jnp.sum(y2, axis=0, keepdims=True), s_ref.shape)
    ss_ref[...] += jnp.broadcast_to(jnp.sum(y2 * y2, axis=0, keepdims=True), ss_ref.shape)


def _layer1(x, cent, w, b, mb, ns):
    m = x.shape[0]
    ct = x.shape[2]
    cout = w.shape[1]
    return pl.pallas_call(
        functools.partial(_k1_body, mb=mb, ns=ns),
        grid=(m // mb,),
        in_specs=[
            pl.BlockSpec((mb, ns, ct), lambda i: (i, 0, 0)),
            pl.BlockSpec((mb, ct), lambda i: (i, 0)),
            pl.BlockSpec(w.shape, lambda i: (0, 0)),
            pl.BlockSpec((1, cout), lambda i: (0, 0)),
        ],
        out_specs=[
            pl.BlockSpec((mb, ns, cout), lambda i: (i, 0, 0)),
            pl.BlockSpec((8, cout), lambda i: (0, 0)),
            pl.BlockSpec((8, cout), lambda i: (0, 0)),
        ],
        out_shape=[
            jax.ShapeDtypeStruct((m, ns, cout), jnp.float32),
            jax.ShapeDtypeStruct((8, cout), jnp.float32),
            jax.ShapeDtypeStruct((8, cout), jnp.float32),
        ],
    )(x, cent, w, b)


def _k1b_body(x_ref, c_ref, b_ref, y_ref, s_ref, ss_ref, *, mb, ns):
    @pl.when(pl.program_id(0) == 0)
    def _():
        s_ref[...] = jnp.zeros_like(s_ref)
        ss_ref[...] = jnp.zeros_like(ss_ref)

    y = x_ref[...] - c_ref[...][:, None, :] + b_ref[...][None]
    y_ref[...] = y
    y2 = y.reshape(mb * ns, y.shape[2])
    s_ref[...] += jnp.broadcast_to(jnp.sum(y2, axis=0, keepdims=True), s_ref.shape)
    ss_ref[...] += jnp.broadcast_to(jnp.sum(y2 * y2, axis=0, keepdims=True), ss_ref.shape)


def _layer1b(x, cent, b, mb, ns):
    m = x.shape[0]
    cout = x.shape[2]
    return pl.pallas_call(
        functools.partial(_k1b_body, mb=mb, ns=ns),
        grid=(m // mb,),
        in_specs=[
            pl.BlockSpec((mb, ns, cout), lambda i: (i, 0, 0)),
            pl.BlockSpec((mb, cout), lambda i: (i, 0)),
            pl.BlockSpec((1, cout), lambda i: (0, 0)),
        ],
        out_specs=[
            pl.BlockSpec((mb, ns, cout), lambda i: (i, 0, 0)),
            pl.BlockSpec((8, cout), lambda i: (0, 0)),
            pl.BlockSpec((8, cout), lambda i: (0, 0)),
        ],
        out_shape=[
            jax.ShapeDtypeStruct((m, ns, cout), jnp.float32),
            jax.ShapeDtypeStruct((8, cout), jnp.float32),
            jax.ShapeDtypeStruct((8, cout), jnp.float32),
        ],
    )(x, cent, b)


def _kmid_body(x_ref, si_ref, ssi_ref, g_ref, be_ref, w_ref, b_ref,
               y_ref, s_ref, ss_ref, *, mb, ns, cnt):
    @pl.when(pl.program_id(0) == 0)
    def _():
        s_ref[...] = jnp.zeros_like(s_ref)
        ss_ref[...] = jnp.zeros_like(ss_ref)

    cin = x_ref.shape[2]
    mean = si_ref[0:1, :] / cnt
    var = ssi_ref[0:1, :] / cnt - mean * mean
    rs = lax.rsqrt(var + 1e-5)
    x2 = x_ref[...].reshape(mb * ns, cin)
    xn = jnp.maximum((x2 - mean) * rs * g_ref[...] + be_ref[...], 0.0)
    y2 = jnp.dot(xn, w_ref[...], preferred_element_type=jnp.float32) + b_ref[...]
    y_ref[...] = y2.reshape(mb, ns, y2.shape[1])
    s_ref[...] += jnp.broadcast_to(jnp.sum(y2, axis=0, keepdims=True), s_ref.shape)
    ss_ref[...] += jnp.broadcast_to(jnp.sum(y2 * y2, axis=0, keepdims=True), ss_ref.shape)


def _layermid(x, si, ssi, g, be, w, b, mb, ns, cnt):
    m = x.shape[0]
    cin = x.shape[2]
    cout = w.shape[1]
    return pl.pallas_call(
        functools.partial(_kmid_body, mb=mb, ns=ns, cnt=float(cnt)),
        grid=(m // mb,),
        in_specs=[
            pl.BlockSpec((mb, ns, cin), lambda i: (i, 0, 0)),
            pl.BlockSpec((8, cin), lambda i: (0, 0)),
            pl.BlockSpec((8, cin), lambda i: (0, 0)),
            pl.BlockSpec((1, cin), lambda i: (0, 0)),
            pl.BlockSpec((1, cin), lambda i: (0, 0)),
            pl.BlockSpec(w.shape, lambda i: (0, 0)),
            pl.BlockSpec((1, cout), lambda i: (0, 0)),
        ],
        out_specs=[
            pl.BlockSpec((mb, ns, cout), lambda i: (i, 0, 0)),
            pl.BlockSpec((8, cout), lambda i: (0, 0)),
            pl.BlockSpec((8, cout), lambda i: (0, 0)),
        ],
        out_shape=[
            jax.ShapeDtypeStruct((m, ns, cout), jnp.float32),
            jax.ShapeDtypeStruct((8, cout), jnp.float32),
            jax.ShapeDtypeStruct((8, cout), jnp.float32),
        ],
    )(x, si, ssi, g.reshape(1, cin), be.reshape(1, cin), w, b)


def _kfin_body(x_ref, si_ref, ssi_ref, g_ref, be_ref, o_ref, *, mb, ns, cnt):
    cin = x_ref.shape[2]
    mean = si_ref[0:1, :] / cnt
    var = ssi_ref[0:1, :] / cnt - mean * mean
    rs = lax.rsqrt(var + 1e-5)
    x2 = x_ref[...].reshape(mb * ns, cin)
    xn = jnp.maximum((x2 - mean) * rs * g_ref[...] + be_ref[...], 0.0)
    o_ref[...] = jnp.max(xn.reshape(mb, ns, cin), axis=1)


def _layerfin(x, si, ssi, g, be, mb, ns, cnt):
    m = x.shape[0]
    cin = x.shape[2]
    return pl.pallas_call(
        functools.partial(_kfin_body, mb=mb, ns=ns, cnt=float(cnt)),
        grid=(m // mb,),
        in_specs=[
            pl.BlockSpec((mb, ns, cin), lambda i: (i, 0, 0)),
            pl.BlockSpec((8, cin), lambda i: (0, 0)),
            pl.BlockSpec((8, cin), lambda i: (0, 0)),
            pl.BlockSpec((1, cin), lambda i: (0, 0)),
            pl.BlockSpec((1, cin), lambda i: (0, 0)),
        ],
        out_specs=pl.BlockSpec((mb, cin), lambda i: (i, 0)),
        out_shape=jax.ShapeDtypeStruct((m, cin), jnp.float32),
    )(x, si, ssi, g.reshape(1, cin), be.reshape(1, cin))


def _mm_body(x_ref, w_ref, o_ref):
    o_ref[...] = jnp.dot(x_ref[...], w_ref[...],
                         preferred_element_type=jnp.float32)


def _mm(x, w):
    return pl.pallas_call(
        _mm_body,
        out_shape=jax.ShapeDtypeStruct((x.shape[0], w.shape[1]), jnp.float32),
    )(x, w)


def _fc_body(x_ref, w1, b1, w2, b2, w3, b3, o_ref):
    h = jnp.dot(x_ref[...], w1[...], preferred_element_type=jnp.float32) + b1[...]
    h = jnp.dot(h, w2[...], preferred_element_type=jnp.float32) + b2[...]
    o_ref[...] = jnp.dot(h, w3[...], preferred_element_type=jnp.float32) + b3[...]


def _fc(x, fc1, fc2, fc3):
    return pl.pallas_call(
        _fc_body,
        out_shape=jax.ShapeDtypeStruct((x.shape[0], fc3[0].shape[1]), jnp.float32),
    )(x, fc1[0], fc1[1].reshape(1, -1), fc2[0], fc2[1].reshape(1, -1),
      fc3[0], fc3[1].reshape(1, -1))


def _msg_scale(gathered, cent, layers, first_is_mm, w1p, mb, ns, m):
    (w1, b1, g1, be1) = layers[0]
    if first_is_mm:
        y1, s1, ss1 = _layer1(gathered, cent, w1p, b1.reshape(1, -1), mb, ns)
    else:
        y1, s1, ss1 = _layer1b(gathered, cent, b1.reshape(1, -1), mb, ns)
    (w2, b2, g2, be2) = layers[1]
    y2, s2, ss2 = _layermid(y1, s1, ss1, g1, be1, w2, b2.reshape(1, -1),
                            mb, ns, m * ns)
    (w3, b3, g3, be3) = layers[2]
    y3, s3, ss3 = _layermid(y2, s2, ss2, g2, be2, w3, b3.reshape(1, -1),
                            mb, ns, m * ns)
    return _layerfin(y3, s3, ss3, g3, be3, mb, ns, m * ns)


def kernel(pointcloud, params):
    pc = pointcloud.astype(jnp.float32)
    xyz = pc[..., :3]
    xyz_t = jnp.transpose(xyz, (0, 2, 1))

    cent1_t = _fps(xyz_t, N, 512)
    cent1_rows = jnp.transpose(cent1_t, (0, 2, 1))
    rank1 = _rank(xyz_t, cent1_rows, (0.1, 0.2, 0.4), np_=512, n=N)
    table1 = jnp.pad(pc.reshape(B * N, 6), ((0, 0), (0, 10)))
    cent1_pad = jnp.pad(cent1_rows.reshape(B * 512, 3), ((0, 0), (0, 13)))
    outs1 = []
    for i, ns in enumerate((16, 32, 128)):
        sck = _make_sc_gather(nrows=B * 512, npts=N, ns=ns, c=16,
                              rows_per_b=512, tab_rows=N, logn=13)
        g = sck(rank1[i].reshape(B * 512, N), table1)
        g = g.reshape(B * 512, ns, 16)
        w1p = jnp.pad(params["sa1"][i][0][0], ((0, 10), (0, 0)))
        mb = 4096 // ns if ns > 16 else 256
        outs1.append(_msg_scale(g, cent1_pad, params["sa1"][i], True, w1p,
                                mb=mb, ns=ns, m=B * 512))
    feats1 = jnp.concatenate([o.reshape(B, 512, o.shape[1]) for o in outs1], -1)

    x2 = jnp.concatenate([cent1_rows, feats1], -1).reshape(B * 512, 323)
    cent2_t = _fps(cent1_t, 512, 128)
    cent2_rows = jnp.transpose(cent2_t, (0, 2, 1))
    rank2 = _rank(cent1_t, cent2_rows, (0.2, 0.4, 0.8), np_=128, n=512)
    cent2_pad = jnp.pad(cent2_rows.reshape(B * 128, 3), ((0, 0), (0, 5)))
    outs2 = []
    for i, ns in enumerate((16, 32, 128)):
        w1 = params["sa2"][i][0][0]
        c1 = w1.shape[1]
        u = _mm(x2, w1)
        w1xyz = jnp.pad(w1[:3], ((0, 5), (0, 0)))
        cu = _mm(cent2_pad, w1xyz)
        sck = _make_sc_gather(nrows=B * 128, npts=512, ns=ns, c=c1,
                              rows_per_b=128, tab_rows=512, logn=10)
        g = sck(rank2[i].reshape(B * 128, 512), u)
        g = g.reshape(B * 128, ns, c1)
        mb = 4096 // ns if ns > 16 else 256
        outs2.append(_msg_scale(g, cu, params["sa2"][i], False, None,
                                mb=mb, ns=ns, m=B * 128))
    feats2 = jnp.concatenate([o.reshape(B, 128, o.shape[1]) for o in outs2], -1)

    x3 = jnp.concatenate([cent2_rows, feats2], -1)
    l3 = params["sa3"]
    zc = jnp.zeros((B, x3.shape[2]), jnp.float32)
    y1, s1, ss1 = _layer1(x3, zc, l3[0][0], l3[0][1].reshape(1, -1), mb=B, ns=128)
    y2, s2, ss2 = _layermid(y1, s1, ss1, l3[0][2], l3[0][3], l3[1][0],
                            l3[1][1].reshape(1, -1), mb=B, ns=128, cnt=B * 128)
    y3, s3, ss3 = _layermid(y2, s2, ss2, l3[1][2], l3[1][3], l3[2][0],
                            l3[2][1].reshape(1, -1), mb=B, ns=128, cnt=B * 128)
    f = _layerfin(y3, s3, ss3, l3[2][2], l3[2][3], mb=B, ns=128, cnt=B * 128)
    return _fc(f, params["fc1"], params["fc2"], params["fc3"])
